# baseline (speedup 1.0000x reference)
import jax
import jax.numpy as jnp
from jax.experimental import pallas as pl
from jax.experimental.pallas import tpu as pltpu

B = 4096
INPUT_DIM = 768
EMBED_DIM = 32
CODEBOOK_SIZE = 8192
N_LAYERS = 3
COMMITMENT_WEIGHT = 0.25
N_CAT_FEATS = 18

TILE_B = 256


def _silu(x):
    return x * jax.nn.sigmoid(x)


def _fused_body(x_ref,
                ew0, ew1, ew2, ew3, eb0, eb1, eb2, eb3,
                cb0, cb1, cb2, g0, g1, g2, c0, c1, c2, gt_ref,
                dw0, dw1, dw2, dw3, db0, db1, db2, db3,
                norm_ref, k1c_ref, k2c_ref, k1r_ref, k2r_ref,
                acc_ref):
    f32 = jnp.float32
    xt = x_ref[...]

    h = _silu(jnp.dot(xt, ew0[...], preferred_element_type=f32) + eb0[...])
    h = _silu(jnp.dot(h, ew1[...], preferred_element_type=f32) + eb1[...])
    h = _silu(jnp.dot(h, ew2[...], preferred_element_type=f32) + eb2[...])
    res = jnp.dot(h, ew3[...], preferred_element_type=f32) + eb3[...]

    qloss = jnp.zeros((TILE_B,), f32)
    emb_sum = jnp.zeros((TILE_B, EMBED_DIM), f32)
    norms = []
    ids_all = []
    sc2 = gt_ref[0, 0]
    colf = jax.lax.broadcasted_iota(
        jnp.int32, (TILE_B, CODEBOOK_SIZE), 1).astype(f32)
    for cb_ref, g_ref, cn_ref in ((cb0, g0, c0), (cb1, g1, c1), (cb2, g2, c2)):
        cross = jax.lax.dot_general(res + res, cb_ref[...],
                                    (((1,), (1,)), ((), ())),
                                    preferred_element_type=f32)
        rn = jnp.sum(res * res, axis=-1, keepdims=True)
        xl = ((rn - cross) + cn_ref[...]) * sc2
        m = jnp.max(xl, axis=-1, keepdims=True)
        w = jnp.exp(xl - m)
        ids = jnp.min(jnp.where(w >= 1.0, colf, float(CODEBOOK_SIZE)),
                      axis=-1).astype(jnp.int32)
        e33 = jnp.dot(w, g_ref[...], preferred_element_type=f32)
        emb = e33[:, :EMBED_DIM] / e33[:, EMBED_DIM:]
        diff = res - emb
        qloss = qloss + (1.0 + COMMITMENT_WEIGHT) * jnp.mean(diff * diff, axis=-1)
        norms.append(jnp.sqrt(jnp.sum(emb * emb, axis=-1)))
        ids_all.append(ids)
        emb_sum = emb_sum + emb
        res = res - emb

    norm_ref[...] = jnp.stack(norms, axis=-1)
    k1 = ids_all[0] * CODEBOOK_SIZE + ids_all[1]
    k2 = ids_all[2]
    k1c_ref[...] = k1[:, None]
    k2c_ref[...] = k2[:, None]
    k1r_ref[...] = k1[None, :]
    k2r_ref[...] = k2[None, :]

    h = _silu(jnp.dot(emb_sum, dw0[...], preferred_element_type=f32) + db0[...])
    h = _silu(jnp.dot(h, dw1[...], preferred_element_type=f32) + db1[...])
    h = _silu(jnp.dot(h, dw2[...], preferred_element_type=f32) + db2[...])
    out = jnp.dot(h, dw3[...], preferred_element_type=f32) + db3[...]

    n = jnp.sqrt(jnp.sum(out * out, axis=-1, keepdims=True))
    x_hat = out / jnp.maximum(n, 1e-12)

    cmask = (jax.lax.broadcasted_iota(jnp.int32, (1, INPUT_DIM), 1)
             < (INPUT_DIM - N_CAT_FEATS)).astype(f32)
    cn2 = jnp.sum(x_hat * x_hat * cmask, axis=-1, keepdims=True)
    cont = x_hat / jnp.maximum(jnp.sqrt(cn2), 1e-12)
    dsq = (cont - xt) ** 2
    sq = jnp.sum(dsq * cmask, axis=-1)

    z = x_hat
    bce_el = (jnp.maximum(z, 0.0) - z * xt + jnp.log1p(jnp.exp(-jnp.abs(z))))
    bce = jnp.sum(bce_el * (1.0 - cmask), axis=-1)

    recon = sq + bce
    part = jnp.stack([jnp.sum(recon.reshape(-1, 128), axis=0),
                      jnp.sum(qloss.reshape(-1, 128), axis=0)], axis=0)

    @pl.when(pl.program_id(0) == 0)
    def _init():
        acc_ref[...] = part

    @pl.when(pl.program_id(0) > 0)
    def _acc():
        acc_ref[...] += part


def _dedup_body(k1c_ref, k2c_ref, k1r_ref, k2r_ref, acc_ref):
    i = pl.program_id(0)
    eq = (k1c_ref[...] == k1r_ref[...]) & (k2c_ref[...] == k2r_ref[...])
    j_idx = jax.lax.broadcasted_iota(jnp.int32, (TILE_B, B), 1)
    i_idx = i * TILE_B + jax.lax.broadcasted_iota(jnp.int32, (TILE_B, B), 0)
    later_dup = jnp.any(eq & (j_idx > i_idx), axis=1)
    flags = 1.0 - later_dup.astype(jnp.float32)
    part = jnp.sum(flags.reshape(-1, 128), axis=0)[None, :]

    @pl.when(i == 0)
    def _init():
        acc_ref[...] = part

    @pl.when(i > 0)
    def _acc():
        acc_ref[...] += part


def kernel(x, enc_Ws, enc_bs, dec_Ws, dec_bs, codebooks, gumbel_t):
    f32 = jnp.float32
    ebs = [b.reshape(1, -1) for b in enc_bs]
    dbs = [b.reshape(1, -1) for b in dec_bs]
    gt = (-1.0 / jnp.asarray(gumbel_t, f32)).reshape(1, 1)
    Cs = [jnp.sum(cb * cb, axis=-1)[None, :] for cb in codebooks]
    Gs = [jnp.concatenate([cb, jnp.ones((CODEBOOK_SIZE, 1), f32)], axis=1)
          for cb in codebooks]

    n_tiles = B // TILE_B
    full = lambda shape: pl.BlockSpec(shape, lambda i: (0, 0))

    in_specs = [pl.BlockSpec((TILE_B, INPUT_DIM), lambda i: (i, 0))]
    in_specs += [full(w.shape) for w in enc_Ws]
    in_specs += [full(b.shape) for b in ebs]
    in_specs += [full(cb.shape) for cb in codebooks]
    in_specs += [full(g.shape) for g in Gs]
    in_specs += [full(c.shape) for c in Cs]
    in_specs += [full((1, 1))]
    in_specs += [full(w.shape) for w in dec_Ws]
    in_specs += [full(b.shape) for b in dbs]

    out_shapes = (
        jax.ShapeDtypeStruct((B, N_LAYERS), f32),
        jax.ShapeDtypeStruct((B, 1), jnp.int32),
        jax.ShapeDtypeStruct((B, 1), jnp.int32),
        jax.ShapeDtypeStruct((1, B), jnp.int32),
        jax.ShapeDtypeStruct((1, B), jnp.int32),
        jax.ShapeDtypeStruct((2, 128), f32),
    )
    out_specs = (
        pl.BlockSpec((TILE_B, N_LAYERS), lambda i: (i, 0)),
        pl.BlockSpec((TILE_B, 1), lambda i: (i, 0)),
        pl.BlockSpec((TILE_B, 1), lambda i: (i, 0)),
        pl.BlockSpec((1, TILE_B), lambda i: (0, i)),
        pl.BlockSpec((1, TILE_B), lambda i: (0, i)),
        pl.BlockSpec((2, 128), lambda i: (0, 0)),
    )

    embs_norm, k1c, k2c, k1r, k2r, acc = pl.pallas_call(
        _fused_body,
        grid=(n_tiles,),
        in_specs=in_specs,
        out_specs=out_specs,
        out_shape=out_shapes,
        compiler_params=pltpu.CompilerParams(
            vmem_limit_bytes=100 * 1024 * 1024),
    )(x, *enc_Ws, *ebs, *codebooks, *Gs, *Cs, gt, *dec_Ws, *dbs)

    recon_sum = jnp.sum(acc[0])
    qloss_sum = jnp.sum(acc[1])
    loss = (recon_sum + qloss_sum) / B

    uacc = pl.pallas_call(
        _dedup_body,
        grid=(n_tiles,),
        in_specs=[
            pl.BlockSpec((TILE_B, 1), lambda i: (i, 0)),
            pl.BlockSpec((TILE_B, 1), lambda i: (i, 0)),
            pl.BlockSpec((1, B), lambda i: (0, 0)),
            pl.BlockSpec((1, B), lambda i: (0, 0)),
        ],
        out_specs=pl.BlockSpec((1, 128), lambda i: (0, 0)),
        out_shape=jax.ShapeDtypeStruct((1, 128), f32),
    )(k1c, k2c, k1r, k2r)
    p_unique = jnp.sum(uacc) / B

    return (loss, recon_sum / B, qloss_sum / B, embs_norm, p_unique)
